# Initial kernel scaffold; baseline (speedup 1.0000x reference)
import jax
import jax.numpy as jnp
from jax.experimental import pallas as pl


def kernel(target_node, cause_node, emotion_node, word_node, target_idx, cause_idx, w_bases, comp, w_root, rgcn_bias, gat_w, att_src, att_dst, gat_bias):
    raise NotImplementedError("write your pallas kernel here")


# speedup vs baseline: 5.4106x; 5.4106x over previous
import numpy as np
import jax
import jax.numpy as jnp
from jax.experimental import pallas as pl

B = 256
C = 8
L = 32
D = 600
N_REL = 8
NUM_BASES = 30
NEG = 0.2
R = 10
GB = 32
DB = 120
RB = 128


def _agg_matrices():
    M = np.zeros((B, N_REL, R, R), np.float64)
    tgt_turn = np.arange(B)
    cs_turn = np.arange(B * C).reshape(B, C)
    for g in range(B):
        edges = []
        for r_ in [0] + [2 + t for t in range(C)]:
            edges.append((r_, 1, 7))
            edges.append((1, r_, 7))
        tt = int(tgt_turn[g])
        for t in range(C):
            d_ = abs(tt - int(cs_turn[g, t]))
            rel = 4 if d_ == 0 else (5 if d_ == 1 else 6)
            edges.append((2 + t, 0, rel))
            edges.append((0, 2 + t, rel))
        for p in range(C):
            for q in range(C):
                if p == q:
                    continue
                d_ = abs(int(cs_turn[g, p]) - int(cs_turn[g, q]))
                fut = int(cs_turn[g, p]) < int(cs_turn[g, q])
                rel = (1 if fut else 0) if d_ == 1 else (3 if fut else 2)
                edges.append((2 + q, 2 + p, rel))
        for dr, sr, rel in edges:
            M[g, rel, dr, sr] += 1.0
    cnt = M.sum(axis=3, keepdims=True)
    M = M / np.maximum(cnt, 1.0)
    return M.reshape(B, N_REL * R, R).astype(np.float32)


_M_ROLES = _agg_matrices()


def _wrel_kernel(comp_ref, wb_ref, out_ref):
    out_ref[...] = jax.lax.dot_general(
        comp_ref[...], wb_ref[...],
        dimension_numbers=(((1,), (0,)), ((), ())),
        preferred_element_type=jnp.float32)


def _gcn_kernel(x_ref, m_ref, wrel_ref, wroot_ref, rb_ref, gw_ref,
                asrc_ref, adst_ref, gb_ref, o1_ref, o2_ref):
    Xf = x_ref[...]
    Xg = Xf.reshape(GB, R, D)
    A = jax.lax.dot_general(
        m_ref[...], Xg,
        dimension_numbers=(((2,), (1,)), ((0,), (0,))),
        preferred_element_type=jnp.float32)
    acc = jnp.dot(Xf, wroot_ref[...],
                  preferred_element_type=jnp.float32) + rb_ref[...]
    for r in range(N_REL):
        Ar = A[:, r * R:(r + 1) * R, :].reshape(GB * R, D)
        acc = acc + jnp.dot(Ar, wrel_ref[r],
                            preferred_element_type=jnp.float32)
    o1_ref[...] = acc
    h = jnp.dot(acc, gw_ref[...], preferred_element_type=jnp.float32)
    asrc = jnp.sum(h * asrc_ref[...], axis=1, keepdims=True)
    adst = jnp.sum(h * adst_ref[...], axis=1, keepdims=True)
    e = adst.reshape(GB, R)[:, :, None] + asrc.reshape(GB, R)[:, None, :]
    e = jnp.where(e > 0, e, NEG * e)
    ee = jnp.exp(e - jnp.max(e, axis=2, keepdims=True))
    alpha = ee / jnp.sum(ee, axis=2, keepdims=True)
    o2 = jax.lax.dot_general(
        alpha, h.reshape(GB, R, D),
        dimension_numbers=(((2,), (1,)), ((0,), (0,))),
        preferred_element_type=jnp.float32)
    o2_ref[...] = o2.reshape(GB * R, D) + gb_ref[...]


def _bcast_kernel(in_ref, out_ref):
    out_ref[...] = jnp.broadcast_to(in_ref[...][:, None, :], (RB, L, D))


def kernel(target_node, cause_node, emotion_node, word_node, target_idx,
           cause_idx, w_bases, comp, w_root, rgcn_bias, gat_w, att_src,
           att_dst, gat_bias):
    x_flat = jnp.concatenate(
        [target_node[:, None, :], emotion_node[:, None, :],
         cause_node.reshape(B, C, D)], axis=1).reshape(B * R, D)
    m_arr = jnp.asarray(_M_ROLES)
    rb2 = rgcn_bias.reshape(1, D)
    gb2 = gat_bias.reshape(1, D)
    as2 = att_src.reshape(1, D)
    ad2 = att_dst.reshape(1, D)

    w_rel = pl.pallas_call(
        _wrel_kernel,
        grid=(D // DB,),
        in_specs=[
            pl.BlockSpec((N_REL, NUM_BASES), lambda i: (0, 0)),
            pl.BlockSpec((NUM_BASES, DB, D), lambda i: (0, i, 0)),
        ],
        out_specs=pl.BlockSpec((N_REL, DB, D), lambda i: (0, i, 0)),
        out_shape=jax.ShapeDtypeStruct((N_REL, D, D), jnp.float32),
    )(comp, w_bases)

    o1, o2 = pl.pallas_call(
        _gcn_kernel,
        grid=(B // GB,),
        in_specs=[
            pl.BlockSpec((GB * R, D), lambda i: (i, 0)),
            pl.BlockSpec((GB, N_REL * R, R), lambda i: (i, 0, 0)),
            pl.BlockSpec((N_REL, D, D), lambda i: (0, 0, 0)),
            pl.BlockSpec((D, D), lambda i: (0, 0)),
            pl.BlockSpec((1, D), lambda i: (0, 0)),
            pl.BlockSpec((D, D), lambda i: (0, 0)),
            pl.BlockSpec((1, D), lambda i: (0, 0)),
            pl.BlockSpec((1, D), lambda i: (0, 0)),
            pl.BlockSpec((1, D), lambda i: (0, 0)),
        ],
        out_specs=[
            pl.BlockSpec((GB * R, D), lambda i: (i, 0)),
            pl.BlockSpec((GB * R, D), lambda i: (i, 0)),
        ],
        out_shape=[jax.ShapeDtypeStruct((B * R, D), jnp.float32)] * 2,
    )(x_flat, m_arr, w_rel, w_root, rb2, gat_w, as2, ad2, gb2)

    out1_roles = o1.reshape(B, R, D)
    out2_roles = o2.reshape(B, R, D)
    out_1 = jnp.concatenate(
        [out1_roles[:, 0], out1_roles[:, 1],
         out1_roles[:, 2:].reshape(B * C, D)], axis=0)
    out_2 = jnp.concatenate(
        [out2_roles[:, 0], out2_roles[:, 1],
         out2_roles[:, 2:].reshape(B * C, D)], axis=0)
    cause2 = out2_roles[:, 2:].reshape(B * C, D)

    out_final = pl.pallas_call(
        _bcast_kernel,
        grid=(B * C // RB,),
        in_specs=[pl.BlockSpec((RB, D), lambda i: (i, 0))],
        out_specs=pl.BlockSpec((RB, L, D), lambda i: (i, 0, 0)),
        out_shape=jax.ShapeDtypeStruct((B * C, L, D), jnp.float32),
    )(cause2)

    return (out_final, out_1, out_2)


# speedup vs baseline: 12.2747x; 2.2686x over previous
import numpy as np
import jax
import jax.numpy as jnp
from jax.experimental import pallas as pl

B = 256
C = 8
L = 32
D = 600
N_REL = 8
NUM_BASES = 30
NEG = 0.2
R = 10
GB = 32
DB = 120
RB = 128


def _agg_matrices():
    M = np.zeros((B, N_REL, R, R), np.float64)
    tgt_turn = np.arange(B)
    cs_turn = np.arange(B * C).reshape(B, C)
    for g in range(B):
        edges = []
        for r_ in [0] + [2 + t for t in range(C)]:
            edges.append((r_, 1, 7))
            edges.append((1, r_, 7))
        tt = int(tgt_turn[g])
        for t in range(C):
            d_ = abs(tt - int(cs_turn[g, t]))
            rel = 4 if d_ == 0 else (5 if d_ == 1 else 6)
            edges.append((2 + t, 0, rel))
            edges.append((0, 2 + t, rel))
        for p in range(C):
            for q in range(C):
                if p == q:
                    continue
                d_ = abs(int(cs_turn[g, p]) - int(cs_turn[g, q]))
                fut = int(cs_turn[g, p]) < int(cs_turn[g, q])
                rel = (1 if fut else 0) if d_ == 1 else (3 if fut else 2)
                edges.append((2 + q, 2 + p, rel))
        for dr, sr, rel in edges:
            M[g, rel, dr, sr] += 1.0
    cnt = M.sum(axis=3, keepdims=True)
    M = M / np.maximum(cnt, 1.0)
    return M.reshape(B, N_REL * R, R).astype(np.float32)


_M_ROLES = _agg_matrices()


def _wrel_kernel(comp_ref, wb_ref, out_ref):
    out_ref[...] = jax.lax.dot_general(
        comp_ref[...], wb_ref[...],
        dimension_numbers=(((1,), (0,)), ((), ())),
        preferred_element_type=jnp.float32)


def _gcn_kernel(t_ref, e_ref, c_ref, m_ref, wrel_ref, wroot_ref, rb_ref,
                gw_ref, asrc_ref, adst_ref, gb_ref,
                o1t_ref, o1e_ref, o1c_ref, o2t_ref, o2e_ref, o2c_ref):
    Xg = jnp.concatenate(
        [t_ref[...][:, None, :], e_ref[...][:, None, :],
         c_ref[...].reshape(GB, C, D)], axis=1)
    Xf = Xg.reshape(GB * R, D)
    A = jax.lax.dot_general(
        m_ref[...], Xg,
        dimension_numbers=(((2,), (1,)), ((0,), (0,))),
        preferred_element_type=jnp.float32)
    acc = jnp.dot(Xf, wroot_ref[...],
                  preferred_element_type=jnp.float32) + rb_ref[...]
    for r in range(N_REL):
        Ar = A[:, r * R:(r + 1) * R, :].reshape(GB * R, D)
        acc = acc + jnp.dot(Ar, wrel_ref[r],
                            preferred_element_type=jnp.float32)
    accg = acc.reshape(GB, R, D)
    o1t_ref[...] = accg[:, 0, :]
    o1e_ref[...] = accg[:, 1, :]
    o1c_ref[...] = accg[:, 2:, :].reshape(GB * C, D)
    h = jnp.dot(acc, gw_ref[...], preferred_element_type=jnp.float32)
    asrc = jnp.sum(h * asrc_ref[...], axis=1, keepdims=True)
    adst = jnp.sum(h * adst_ref[...], axis=1, keepdims=True)
    e = adst.reshape(GB, R)[:, :, None] + asrc.reshape(GB, R)[:, None, :]
    e = jnp.where(e > 0, e, NEG * e)
    ee = jnp.exp(e - jnp.max(e, axis=2, keepdims=True))
    alpha = ee / jnp.sum(ee, axis=2, keepdims=True)
    o2 = jax.lax.dot_general(
        alpha, h.reshape(GB, R, D),
        dimension_numbers=(((2,), (1,)), ((0,), (0,))),
        preferred_element_type=jnp.float32)
    o2 = o2 + gb_ref[...]
    o2t_ref[...] = o2[:, 0, :]
    o2e_ref[...] = o2[:, 1, :]
    o2c_ref[...] = o2[:, 2:, :].reshape(GB * C, D)


def _bcast_kernel(in_ref, out_ref):
    xt = in_ref[...].T
    out_ref[...] = jnp.broadcast_to(xt[None, :, :], (L, D, RB))


def kernel(target_node, cause_node, emotion_node, word_node, target_idx,
           cause_idx, w_bases, comp, w_root, rgcn_bias, gat_w, att_src,
           att_dst, gat_bias):
    m_arr = jnp.asarray(_M_ROLES)
    rb2 = rgcn_bias.reshape(1, D)
    gb2 = gat_bias.reshape(1, D)
    as2 = att_src.reshape(1, D)
    ad2 = att_dst.reshape(1, D)

    w_rel = pl.pallas_call(
        _wrel_kernel,
        grid=(D // DB,),
        in_specs=[
            pl.BlockSpec((N_REL, NUM_BASES), lambda i: (0, 0)),
            pl.BlockSpec((NUM_BASES, DB, D), lambda i: (0, i, 0)),
        ],
        out_specs=pl.BlockSpec((N_REL, DB, D), lambda i: (0, i, 0)),
        out_shape=jax.ShapeDtypeStruct((N_REL, D, D), jnp.float32),
    )(comp, w_bases)

    sec = jax.ShapeDtypeStruct((B, D), jnp.float32)
    secc = jax.ShapeDtypeStruct((B * C, D), jnp.float32)
    o1t, o1e, o1c, o2t, o2e, o2c = pl.pallas_call(
        _gcn_kernel,
        grid=(B // GB,),
        in_specs=[
            pl.BlockSpec((GB, D), lambda i: (i, 0)),
            pl.BlockSpec((GB, D), lambda i: (i, 0)),
            pl.BlockSpec((GB * C, D), lambda i: (i, 0)),
            pl.BlockSpec((GB, N_REL * R, R), lambda i: (i, 0, 0)),
            pl.BlockSpec((N_REL, D, D), lambda i: (0, 0, 0)),
            pl.BlockSpec((D, D), lambda i: (0, 0)),
            pl.BlockSpec((1, D), lambda i: (0, 0)),
            pl.BlockSpec((D, D), lambda i: (0, 0)),
            pl.BlockSpec((1, D), lambda i: (0, 0)),
            pl.BlockSpec((1, D), lambda i: (0, 0)),
            pl.BlockSpec((1, D), lambda i: (0, 0)),
        ],
        out_specs=[
            pl.BlockSpec((GB, D), lambda i: (i, 0)),
            pl.BlockSpec((GB, D), lambda i: (i, 0)),
            pl.BlockSpec((GB * C, D), lambda i: (i, 0)),
            pl.BlockSpec((GB, D), lambda i: (i, 0)),
            pl.BlockSpec((GB, D), lambda i: (i, 0)),
            pl.BlockSpec((GB * C, D), lambda i: (i, 0)),
        ],
        out_shape=[sec, sec, secc, sec, sec, secc],
    )(target_node, emotion_node, cause_node, m_arr, w_rel, w_root, rb2,
      gat_w, as2, ad2, gb2)

    out_1 = jnp.concatenate([o1t, o1e, o1c], axis=0)
    out_2 = jnp.concatenate([o2t, o2e, o2c], axis=0)

    out_final_t = pl.pallas_call(
        _bcast_kernel,
        grid=(B * C // RB,),
        in_specs=[pl.BlockSpec((RB, D), lambda i: (i, 0))],
        out_specs=pl.BlockSpec((L, D, RB), lambda i: (0, 0, i)),
        out_shape=jax.ShapeDtypeStruct((L, D, B * C), jnp.float32),
    )(o2c)

    out_final = jnp.transpose(out_final_t, (2, 0, 1))
    return (out_final, out_1, out_2)


# speedup vs baseline: 14.5341x; 1.1841x over previous
import numpy as np
import jax
import jax.numpy as jnp
from jax.experimental import pallas as pl

B = 256
C = 8
L = 32
D = 600
N_REL = 8
NUM_BASES = 30
NEG = 0.2
R = 10
GB = 16
DB = 120


def _agg_matrices():
    M = np.zeros((B, N_REL, R, R), np.float64)
    tgt_turn = np.arange(B)
    cs_turn = np.arange(B * C).reshape(B, C)
    for g in range(B):
        edges = []
        for r_ in [0] + [2 + t for t in range(C)]:
            edges.append((r_, 1, 7))
            edges.append((1, r_, 7))
        tt = int(tgt_turn[g])
        for t in range(C):
            d_ = abs(tt - int(cs_turn[g, t]))
            rel = 4 if d_ == 0 else (5 if d_ == 1 else 6)
            edges.append((2 + t, 0, rel))
            edges.append((0, 2 + t, rel))
        for p in range(C):
            for q in range(C):
                if p == q:
                    continue
                d_ = abs(int(cs_turn[g, p]) - int(cs_turn[g, q]))
                fut = int(cs_turn[g, p]) < int(cs_turn[g, q])
                rel = (1 if fut else 0) if d_ == 1 else (3 if fut else 2)
                edges.append((2 + q, 2 + p, rel))
        for dr, sr, rel in edges:
            M[g, rel, dr, sr] += 1.0
    cnt = M.sum(axis=3, keepdims=True)
    M = M / np.maximum(cnt, 1.0)
    return M.reshape(B, N_REL * R, R).astype(np.float32)


_M_ROLES = _agg_matrices()


def _wrel_kernel(comp_ref, wb_ref, out_ref):
    out_ref[...] = jax.lax.dot_general(
        comp_ref[...], wb_ref[...],
        dimension_numbers=(((1,), (0,)), ((), ())),
        preferred_element_type=jnp.float32)


def _gcn_kernel(t_ref, e_ref, c_ref, m_ref, wrel_ref, wroot_ref, rb_ref,
                gw_ref, asrc_ref, adst_ref, gb_ref,
                o1t_ref, o1e_ref, o1c_ref, o2t_ref, o2e_ref, o2c_ref,
                oft_ref):
    Xg = jnp.concatenate(
        [t_ref[...][:, None, :], e_ref[...][:, None, :],
         c_ref[...].reshape(GB, C, D)], axis=1)
    Xf = Xg.reshape(GB * R, D)
    A = jax.lax.dot_general(
        m_ref[...], Xg,
        dimension_numbers=(((2,), (1,)), ((0,), (0,))),
        preferred_element_type=jnp.float32)
    acc = jnp.dot(Xf, wroot_ref[...],
                  preferred_element_type=jnp.float32) + rb_ref[...]
    for r in range(N_REL):
        Ar = A[:, r * R:(r + 1) * R, :].reshape(GB * R, D)
        acc = acc + jnp.dot(Ar, wrel_ref[r],
                            preferred_element_type=jnp.float32)
    accg = acc.reshape(GB, R, D)
    o1t_ref[...] = accg[:, 0, :]
    o1e_ref[...] = accg[:, 1, :]
    o1c_ref[...] = accg[:, 2:, :].reshape(GB * C, D)
    h = jnp.dot(acc, gw_ref[...], preferred_element_type=jnp.float32)
    asrc = jnp.sum(h * asrc_ref[...], axis=1, keepdims=True)
    adst = jnp.sum(h * adst_ref[...], axis=1, keepdims=True)
    e = adst.reshape(GB, R)[:, :, None] + asrc.reshape(GB, R)[:, None, :]
    e = jnp.where(e > 0, e, NEG * e)
    ee = jnp.exp(e - jnp.max(e, axis=2, keepdims=True))
    alpha = ee / jnp.sum(ee, axis=2, keepdims=True)
    o2 = jax.lax.dot_general(
        alpha, h.reshape(GB, R, D),
        dimension_numbers=(((2,), (1,)), ((0,), (0,))),
        preferred_element_type=jnp.float32)
    o2 = o2 + gb_ref[...]
    o2t_ref[...] = o2[:, 0, :]
    o2e_ref[...] = o2[:, 1, :]
    cz = o2[:, 2:, :].reshape(GB * C, D)
    o2c_ref[...] = cz
    oft_ref[...] = jnp.broadcast_to(cz.T[None, :, :], (L, D, GB * C))


def kernel(target_node, cause_node, emotion_node, word_node, target_idx,
           cause_idx, w_bases, comp, w_root, rgcn_bias, gat_w, att_src,
           att_dst, gat_bias):
    m_arr = jnp.asarray(_M_ROLES)
    rb2 = rgcn_bias.reshape(1, D)
    gb2 = gat_bias.reshape(1, D)
    as2 = att_src.reshape(1, D)
    ad2 = att_dst.reshape(1, D)

    w_rel = pl.pallas_call(
        _wrel_kernel,
        grid=(D // DB,),
        in_specs=[
            pl.BlockSpec((N_REL, NUM_BASES), lambda i: (0, 0)),
            pl.BlockSpec((NUM_BASES, DB, D), lambda i: (0, i, 0)),
        ],
        out_specs=pl.BlockSpec((N_REL, DB, D), lambda i: (0, i, 0)),
        out_shape=jax.ShapeDtypeStruct((N_REL, D, D), jnp.float32),
    )(comp, w_bases)

    sec = jax.ShapeDtypeStruct((B, D), jnp.float32)
    secc = jax.ShapeDtypeStruct((B * C, D), jnp.float32)
    oft = jax.ShapeDtypeStruct((L, D, B * C), jnp.float32)
    o1t, o1e, o1c, o2t, o2e, o2c, out_final_t = pl.pallas_call(
        _gcn_kernel,
        grid=(B // GB,),
        in_specs=[
            pl.BlockSpec((GB, D), lambda i: (i, 0)),
            pl.BlockSpec((GB, D), lambda i: (i, 0)),
            pl.BlockSpec((GB * C, D), lambda i: (i, 0)),
            pl.BlockSpec((GB, N_REL * R, R), lambda i: (i, 0, 0)),
            pl.BlockSpec((N_REL, D, D), lambda i: (0, 0, 0)),
            pl.BlockSpec((D, D), lambda i: (0, 0)),
            pl.BlockSpec((1, D), lambda i: (0, 0)),
            pl.BlockSpec((D, D), lambda i: (0, 0)),
            pl.BlockSpec((1, D), lambda i: (0, 0)),
            pl.BlockSpec((1, D), lambda i: (0, 0)),
            pl.BlockSpec((1, D), lambda i: (0, 0)),
        ],
        out_specs=[
            pl.BlockSpec((GB, D), lambda i: (i, 0)),
            pl.BlockSpec((GB, D), lambda i: (i, 0)),
            pl.BlockSpec((GB * C, D), lambda i: (i, 0)),
            pl.BlockSpec((GB, D), lambda i: (i, 0)),
            pl.BlockSpec((GB, D), lambda i: (i, 0)),
            pl.BlockSpec((GB * C, D), lambda i: (i, 0)),
            pl.BlockSpec((L, D, GB * C), lambda i: (0, 0, i)),
        ],
        out_shape=[sec, sec, secc, sec, sec, secc, oft],
    )(target_node, emotion_node, cause_node, m_arr, w_rel, w_root, rb2,
      gat_w, as2, ad2, gb2)

    out_1 = jnp.concatenate([o1t, o1e, o1c], axis=0)
    out_2 = jnp.concatenate([o2t, o2e, o2c], axis=0)
    out_final = jnp.transpose(out_final_t, (2, 0, 1))
    return (out_final, out_1, out_2)


# speedup vs baseline: 14.7092x; 1.0120x over previous
import numpy as np
import jax
import jax.numpy as jnp
from jax.experimental import pallas as pl

B = 256
C = 8
L = 32
D = 600
N_REL = 8
NUM_BASES = 30
NEG = 0.2
R = 10
GB = 16
DB = 120


def _agg_matrices():
    M = np.zeros((B, N_REL, R, R), np.float64)
    tgt_turn = np.arange(B)
    cs_turn = np.arange(B * C).reshape(B, C)
    for g in range(B):
        edges = []
        for r_ in [0] + [2 + t for t in range(C)]:
            edges.append((r_, 1, 7))
            edges.append((1, r_, 7))
        tt = int(tgt_turn[g])
        for t in range(C):
            d_ = abs(tt - int(cs_turn[g, t]))
            rel = 4 if d_ == 0 else (5 if d_ == 1 else 6)
            edges.append((2 + t, 0, rel))
            edges.append((0, 2 + t, rel))
        for p in range(C):
            for q in range(C):
                if p == q:
                    continue
                d_ = abs(int(cs_turn[g, p]) - int(cs_turn[g, q]))
                fut = int(cs_turn[g, p]) < int(cs_turn[g, q])
                rel = (1 if fut else 0) if d_ == 1 else (3 if fut else 2)
                edges.append((2 + q, 2 + p, rel))
        for dr, sr, rel in edges:
            M[g, rel, dr, sr] += 1.0
    cnt = M.sum(axis=3, keepdims=True)
    M = M / np.maximum(cnt, 1.0)
    return M.reshape(B, N_REL * R, R).astype(np.float32)


_M_ROLES = _agg_matrices()


def _wrel_kernel(comp_ref, wb_ref, out_ref):
    out_ref[...] = jax.lax.dot_general(
        comp_ref[...], wb_ref[...],
        dimension_numbers=(((1,), (0,)), ((), ())),
        preferred_element_type=jnp.float32).astype(jnp.bfloat16)


def _gcn_kernel(t_ref, e_ref, c_ref, m_ref, wrel_ref, wroot_ref, rb_ref,
                gw_ref, asrc_ref, adst_ref, gb_ref,
                o1t_ref, o1e_ref, o1c_ref, o2t_ref, o2e_ref, o2c_ref,
                oft_ref):
    Xg = jnp.concatenate(
        [t_ref[...][:, None, :], e_ref[...][:, None, :],
         c_ref[...].reshape(GB, C, D)], axis=1)
    Xf = Xg.reshape(GB * R, D)
    Xb = Xf.astype(jnp.bfloat16)
    A = jax.lax.dot_general(
        m_ref[...], Xb.reshape(GB, R, D),
        dimension_numbers=(((2,), (1,)), ((0,), (0,))),
        preferred_element_type=jnp.float32
        ).astype(jnp.bfloat16)
    acc = jnp.dot(Xb, wroot_ref[...],
                  preferred_element_type=jnp.float32) + rb_ref[...]
    for r in range(N_REL):
        Ar = A[:, r * R:(r + 1) * R, :].reshape(GB * R, D)
        acc = acc + jnp.dot(Ar, wrel_ref[r],
                            preferred_element_type=jnp.float32)
    accg = acc.reshape(GB, R, D)
    o1t_ref[...] = accg[:, 0, :]
    o1e_ref[...] = accg[:, 1, :]
    o1c_ref[...] = accg[:, 2:, :].reshape(GB * C, D)
    h = jnp.dot(acc.astype(jnp.bfloat16), gw_ref[...],
                preferred_element_type=jnp.float32)
    asrc = jnp.sum(h * asrc_ref[...], axis=1, keepdims=True)
    adst = jnp.sum(h * adst_ref[...], axis=1, keepdims=True)
    e = adst.reshape(GB, R)[:, :, None] + asrc.reshape(GB, R)[:, None, :]
    e = jnp.where(e > 0, e, NEG * e)
    ee = jnp.exp(e - jnp.max(e, axis=2, keepdims=True))
    alpha = ee / jnp.sum(ee, axis=2, keepdims=True)
    o2 = jax.lax.dot_general(
        alpha.astype(jnp.bfloat16), h.reshape(GB, R, D).astype(jnp.bfloat16),
        dimension_numbers=(((2,), (1,)), ((0,), (0,))),
        preferred_element_type=jnp.float32)
    o2 = o2 + gb_ref[...]
    o2t_ref[...] = o2[:, 0, :]
    o2e_ref[...] = o2[:, 1, :]
    cz = o2[:, 2:, :].reshape(GB * C, D)
    o2c_ref[...] = cz
    oft_ref[...] = jnp.broadcast_to(cz.T[None, :, :], (L, D, GB * C))


def kernel(target_node, cause_node, emotion_node, word_node, target_idx,
           cause_idx, w_bases, comp, w_root, rgcn_bias, gat_w, att_src,
           att_dst, gat_bias):
    m_arr = jnp.asarray(_M_ROLES).astype(jnp.bfloat16)
    wroot_b = w_root.astype(jnp.bfloat16)
    gat_w_b = gat_w.astype(jnp.bfloat16)
    rb2 = rgcn_bias.reshape(1, D)
    gb2 = gat_bias.reshape(1, D)
    as2 = att_src.reshape(1, D)
    ad2 = att_dst.reshape(1, D)

    w_rel = pl.pallas_call(
        _wrel_kernel,
        grid=(D // DB,),
        in_specs=[
            pl.BlockSpec((N_REL, NUM_BASES), lambda i: (0, 0)),
            pl.BlockSpec((NUM_BASES, DB, D), lambda i: (0, i, 0)),
        ],
        out_specs=pl.BlockSpec((N_REL, DB, D), lambda i: (0, i, 0)),
        out_shape=jax.ShapeDtypeStruct((N_REL, D, D), jnp.bfloat16),
    )(comp, w_bases)

    sec = jax.ShapeDtypeStruct((B, D), jnp.float32)
    secc = jax.ShapeDtypeStruct((B * C, D), jnp.float32)
    oft = jax.ShapeDtypeStruct((L, D, B * C), jnp.float32)
    o1t, o1e, o1c, o2t, o2e, o2c, out_final_t = pl.pallas_call(
        _gcn_kernel,
        grid=(B // GB,),
        in_specs=[
            pl.BlockSpec((GB, D), lambda i: (i, 0)),
            pl.BlockSpec((GB, D), lambda i: (i, 0)),
            pl.BlockSpec((GB * C, D), lambda i: (i, 0)),
            pl.BlockSpec((GB, N_REL * R, R), lambda i: (i, 0, 0)),
            pl.BlockSpec((N_REL, D, D), lambda i: (0, 0, 0)),
            pl.BlockSpec((D, D), lambda i: (0, 0)),
            pl.BlockSpec((1, D), lambda i: (0, 0)),
            pl.BlockSpec((D, D), lambda i: (0, 0)),
            pl.BlockSpec((1, D), lambda i: (0, 0)),
            pl.BlockSpec((1, D), lambda i: (0, 0)),
            pl.BlockSpec((1, D), lambda i: (0, 0)),
        ],
        out_specs=[
            pl.BlockSpec((GB, D), lambda i: (i, 0)),
            pl.BlockSpec((GB, D), lambda i: (i, 0)),
            pl.BlockSpec((GB * C, D), lambda i: (i, 0)),
            pl.BlockSpec((GB, D), lambda i: (i, 0)),
            pl.BlockSpec((GB, D), lambda i: (i, 0)),
            pl.BlockSpec((GB * C, D), lambda i: (i, 0)),
            pl.BlockSpec((L, D, GB * C), lambda i: (0, 0, i)),
        ],
        out_shape=[sec, sec, secc, sec, sec, secc, oft],
    )(target_node, emotion_node, cause_node, m_arr, w_rel, wroot_b, rb2,
      gat_w_b, as2, ad2, gb2)

    out_1 = jnp.concatenate([o1t, o1e, o1c], axis=0)
    out_2 = jnp.concatenate([o2t, o2e, o2c], axis=0)
    out_final = jnp.transpose(out_final_t, (2, 0, 1))
    return (out_final, out_1, out_2)


# speedup vs baseline: 14.9491x; 1.0163x over previous
import numpy as np
import jax
import jax.numpy as jnp
from jax.experimental import pallas as pl

B = 256
C = 8
L = 32
D = 600
N_REL = 8
NUM_BASES = 30
NEG = 0.2
R = 10
GB = 16
NB = GB * R
DB = 120


def _agg_matrices():
    M = np.zeros((B, N_REL, R, R), np.float64)
    tgt_turn = np.arange(B)
    cs_turn = np.arange(B * C).reshape(B, C)
    for g in range(B):
        edges = []
        for r_ in [0] + [2 + t for t in range(C)]:
            edges.append((r_, 1, 7))
            edges.append((1, r_, 7))
        tt = int(tgt_turn[g])
        for t in range(C):
            d_ = abs(tt - int(cs_turn[g, t]))
            rel = 4 if d_ == 0 else (5 if d_ == 1 else 6)
            edges.append((2 + t, 0, rel))
            edges.append((0, 2 + t, rel))
        for p in range(C):
            for q in range(C):
                if p == q:
                    continue
                d_ = abs(int(cs_turn[g, p]) - int(cs_turn[g, q]))
                fut = int(cs_turn[g, p]) < int(cs_turn[g, q])
                rel = (1 if fut else 0) if d_ == 1 else (3 if fut else 2)
                edges.append((2 + q, 2 + p, rel))
        for dr, sr, rel in edges:
            M[g, rel, dr, sr] += 1.0
    cnt = M.sum(axis=3, keepdims=True)
    return (M / np.maximum(cnt, 1.0)).astype(np.float32)


def _blockdiag_mats():
    M = _agg_matrices()
    out = np.zeros((2, N_REL, NB, NB), np.float32)
    for v in range(2):
        for r in range(N_REL):
            for k in range(GB):
                g = k if v == 0 else GB + k
                out[v, r, k * R:(k + 1) * R, k * R:(k + 1) * R] = M[g, r]
    return out


_M_BIG = _blockdiag_mats()


def _wrel_kernel(comp_ref, wb_ref, wr_ref, gw_ref,
                 out_ref, wrb_ref, gwb_ref):
    out_ref[...] = jax.lax.dot_general(
        comp_ref[...], wb_ref[...],
        dimension_numbers=(((1,), (0,)), ((), ())),
        preferred_element_type=jnp.float32).astype(jnp.bfloat16)
    wrb_ref[...] = wr_ref[...].astype(jnp.bfloat16)
    gwb_ref[...] = gw_ref[...].astype(jnp.bfloat16)


def _gcn_kernel(t_ref, e_ref, c_ref, mb_ref, wrel_ref, wroot_ref, gw_ref,
                vec_ref, o1t_ref, o1e_ref, o1c_ref, o2t_ref, o2e_ref,
                o2c_ref, oft_ref):
    Xg = jnp.concatenate(
        [t_ref[...][:, None, :], e_ref[...][:, None, :],
         c_ref[...].reshape(GB, C, D)], axis=1)
    Xb = Xg.reshape(NB, D).astype(jnp.bfloat16)
    rb = vec_ref[0:1, :]
    gb = vec_ref[1:2, :]
    asv = vec_ref[2:3, :]
    adv = vec_ref[3:4, :]
    acc = jnp.dot(Xb, wroot_ref[...],
                  preferred_element_type=jnp.float32) + rb
    for r in range(N_REL):
        Yr = jnp.dot(Xb, wrel_ref[r],
                     preferred_element_type=jnp.float32).astype(jnp.bfloat16)
        acc = acc + jnp.dot(mb_ref[0, r], Yr,
                            preferred_element_type=jnp.float32)
    accg = acc.reshape(GB, R, D)
    o1t_ref[...] = accg[:, 0, :]
    o1e_ref[...] = accg[:, 1, :]
    o1c_ref[...] = accg[:, 2:, :].reshape(GB * C, D).T
    h = jnp.dot(acc.astype(jnp.bfloat16), gw_ref[...],
                preferred_element_type=jnp.float32)
    asrc = jnp.sum(h * asv, axis=1, keepdims=True)
    adst = jnp.sum(h * adv, axis=1, keepdims=True)
    e = adst.reshape(GB, R)[:, :, None] + asrc.reshape(GB, R)[:, None, :]
    e = jnp.where(e > 0, e, NEG * e)
    ee = jnp.exp(e - jnp.max(e, axis=2, keepdims=True))
    alpha = ee / jnp.sum(ee, axis=2, keepdims=True)
    o2 = jax.lax.dot_general(
        alpha.astype(jnp.bfloat16), h.reshape(GB, R, D).astype(jnp.bfloat16),
        dimension_numbers=(((2,), (1,)), ((0,), (0,))),
        preferred_element_type=jnp.float32)
    o2 = o2 + gb
    o2t_ref[...] = o2[:, 0, :]
    o2e_ref[...] = o2[:, 1, :]
    czT = o2[:, 2:, :].reshape(GB * C, D).T
    o2c_ref[...] = czT
    oft_ref[...] = jnp.broadcast_to(czT[None, :, :], (L, D, GB * C))


def kernel(target_node, cause_node, emotion_node, word_node, target_idx,
           cause_idx, w_bases, comp, w_root, rgcn_bias, gat_w, att_src,
           att_dst, gat_bias):
    mb_arr = jnp.asarray(_M_BIG).astype(jnp.bfloat16)
    vecs = jnp.stack([rgcn_bias, gat_bias, att_src, att_dst], axis=0)

    w_rel, wroot_b, gatw_b = pl.pallas_call(
        _wrel_kernel,
        grid=(D // DB,),
        in_specs=[
            pl.BlockSpec((N_REL, NUM_BASES), lambda i: (0, 0)),
            pl.BlockSpec((NUM_BASES, DB, D), lambda i: (0, i, 0)),
            pl.BlockSpec((DB, D), lambda i: (i, 0)),
            pl.BlockSpec((DB, D), lambda i: (i, 0)),
        ],
        out_specs=[
            pl.BlockSpec((N_REL, DB, D), lambda i: (0, i, 0)),
            pl.BlockSpec((DB, D), lambda i: (i, 0)),
            pl.BlockSpec((DB, D), lambda i: (i, 0)),
        ],
        out_shape=[
            jax.ShapeDtypeStruct((N_REL, D, D), jnp.bfloat16),
            jax.ShapeDtypeStruct((D, D), jnp.bfloat16),
            jax.ShapeDtypeStruct((D, D), jnp.bfloat16),
        ],
    )(comp, w_bases, w_root, gat_w)

    sec = jax.ShapeDtypeStruct((B, D), jnp.float32)
    secc = jax.ShapeDtypeStruct((D, B * C), jnp.float32)
    oft = jax.ShapeDtypeStruct((L, D, B * C), jnp.float32)
    o1t, o1e, o1cT, o2t, o2e, o2cT, out_final_t = pl.pallas_call(
        _gcn_kernel,
        grid=(B // GB,),
        in_specs=[
            pl.BlockSpec((GB, D), lambda i: (i, 0)),
            pl.BlockSpec((GB, D), lambda i: (i, 0)),
            pl.BlockSpec((GB * C, D), lambda i: (i, 0)),
            pl.BlockSpec((1, N_REL, NB, NB),
                         lambda i: (jnp.minimum(i, 1), 0, 0, 0)),
            pl.BlockSpec((N_REL, D, D), lambda i: (0, 0, 0)),
            pl.BlockSpec((D, D), lambda i: (0, 0)),
            pl.BlockSpec((D, D), lambda i: (0, 0)),
            pl.BlockSpec((4, D), lambda i: (0, 0)),
        ],
        out_specs=[
            pl.BlockSpec((GB, D), lambda i: (i, 0)),
            pl.BlockSpec((GB, D), lambda i: (i, 0)),
            pl.BlockSpec((D, GB * C), lambda i: (0, i)),
            pl.BlockSpec((GB, D), lambda i: (i, 0)),
            pl.BlockSpec((GB, D), lambda i: (i, 0)),
            pl.BlockSpec((D, GB * C), lambda i: (0, i)),
            pl.BlockSpec((L, D, GB * C), lambda i: (0, 0, i)),
        ],
        out_shape=[sec, sec, secc, sec, sec, secc, oft],
    )(target_node, emotion_node, cause_node, mb_arr, w_rel, wroot_b,
      gatw_b, vecs)

    out_1 = jnp.concatenate([o1t, o1e, o1cT.T], axis=0)
    out_2 = jnp.concatenate([o2t, o2e, o2cT.T], axis=0)
    out_final = jnp.transpose(out_final_t, (2, 0, 1))
    return (out_final, out_1, out_2)


# speedup vs baseline: 16.3045x; 1.0907x over previous
import numpy as np
import jax
import jax.numpy as jnp
from jax.experimental import pallas as pl

B = 256
C = 8
L = 32
D = 600
N_REL = 8
NUM_BASES = 30
NEG = 0.2
R = 10
GB = 16
NB = GB * R
DB = 120


def _agg_matrices():
    M = np.zeros((B, N_REL, R, R), np.float64)
    tgt_turn = np.arange(B)
    cs_turn = np.arange(B * C).reshape(B, C)
    for g in range(B):
        edges = []
        for r_ in [0] + [2 + t for t in range(C)]:
            edges.append((r_, 1, 7))
            edges.append((1, r_, 7))
        tt = int(tgt_turn[g])
        for t in range(C):
            d_ = abs(tt - int(cs_turn[g, t]))
            rel = 4 if d_ == 0 else (5 if d_ == 1 else 6)
            edges.append((2 + t, 0, rel))
            edges.append((0, 2 + t, rel))
        for p in range(C):
            for q in range(C):
                if p == q:
                    continue
                d_ = abs(int(cs_turn[g, p]) - int(cs_turn[g, q]))
                fut = int(cs_turn[g, p]) < int(cs_turn[g, q])
                rel = (1 if fut else 0) if d_ == 1 else (3 if fut else 2)
                edges.append((2 + q, 2 + p, rel))
        for dr, sr, rel in edges:
            M[g, rel, dr, sr] += 1.0
    cnt = M.sum(axis=3, keepdims=True)
    return (M / np.maximum(cnt, 1.0)).astype(np.float32)


_M_ROLES = _agg_matrices().reshape(B, N_REL * R, R)


def _wrel_kernel(comp_ref, wb_ref, wr_ref, gw_ref,
                 out_ref, wrb_ref, gwb_ref):
    out_ref[...] = jax.lax.dot_general(
        comp_ref[...], wb_ref[...],
        dimension_numbers=(((1,), (0,)), ((), ())),
        preferred_element_type=jnp.float32).astype(jnp.bfloat16)
    wrb_ref[...] = wr_ref[...].astype(jnp.bfloat16)
    gwb_ref[...] = gw_ref[...].astype(jnp.bfloat16)


def _gcn_kernel(t_ref, e_ref, c_ref, mb_ref, wrel_ref, wroot_ref, gw_ref,
                vec_ref, o1t_ref, o1e_ref, o1c_ref, o2t_ref, o2e_ref,
                o2c_ref, oft_ref):
    Xg = jnp.concatenate(
        [t_ref[...][:, None, :], e_ref[...][:, None, :],
         c_ref[...].reshape(GB, C, D)], axis=1)
    Xb = Xg.reshape(NB, D).astype(jnp.bfloat16)
    rb = vec_ref[0:1, :]
    gb = vec_ref[1:2, :]
    asv = vec_ref[2:3, :]
    adv = vec_ref[3:4, :]
    A = jax.lax.dot_general(
        mb_ref[...], Xb.reshape(GB, R, D),
        dimension_numbers=(((2,), (1,)), ((0,), (0,))),
        preferred_element_type=jnp.float32
        ).astype(jnp.bfloat16)
    acc = jnp.dot(Xb, wroot_ref[...],
                  preferred_element_type=jnp.float32) + rb
    for r in range(N_REL):
        Ar = A[:, r * R:(r + 1) * R, :].reshape(NB, D)
        acc = acc + jnp.dot(Ar, wrel_ref[r],
                            preferred_element_type=jnp.float32)
    accg = acc.reshape(GB, R, D)
    o1t_ref[...] = accg[:, 0, :]
    o1e_ref[...] = accg[:, 1, :]
    o1c_ref[...] = accg[:, 2:, :].reshape(GB * C, D).T
    h = jnp.dot(acc.astype(jnp.bfloat16), gw_ref[...],
                preferred_element_type=jnp.float32)
    asrc = jnp.sum(h * asv, axis=1, keepdims=True)
    adst = jnp.sum(h * adv, axis=1, keepdims=True)
    e = adst.reshape(GB, R)[:, :, None] + asrc.reshape(GB, R)[:, None, :]
    e = jnp.where(e > 0, e, NEG * e)
    ee = jnp.exp(e - jnp.max(e, axis=2, keepdims=True))
    alpha = ee / jnp.sum(ee, axis=2, keepdims=True)
    o2 = jax.lax.dot_general(
        alpha.astype(jnp.bfloat16), h.reshape(GB, R, D).astype(jnp.bfloat16),
        dimension_numbers=(((2,), (1,)), ((0,), (0,))),
        preferred_element_type=jnp.float32)
    o2 = o2 + gb
    o2t_ref[...] = o2[:, 0, :]
    o2e_ref[...] = o2[:, 1, :]
    czT = o2[:, 2:, :].reshape(GB * C, D).T
    o2c_ref[...] = czT
    oft_ref[...] = jnp.broadcast_to(czT[None, :, :], (L, D, GB * C))


def kernel(target_node, cause_node, emotion_node, word_node, target_idx,
           cause_idx, w_bases, comp, w_root, rgcn_bias, gat_w, att_src,
           att_dst, gat_bias):
    mb_arr = jnp.asarray(_M_ROLES).astype(jnp.bfloat16)
    vecs = jnp.stack([rgcn_bias, gat_bias, att_src, att_dst], axis=0)

    w_rel, wroot_b, gatw_b = pl.pallas_call(
        _wrel_kernel,
        grid=(D // DB,),
        in_specs=[
            pl.BlockSpec((N_REL, NUM_BASES), lambda i: (0, 0)),
            pl.BlockSpec((NUM_BASES, DB, D), lambda i: (0, i, 0)),
            pl.BlockSpec((DB, D), lambda i: (i, 0)),
            pl.BlockSpec((DB, D), lambda i: (i, 0)),
        ],
        out_specs=[
            pl.BlockSpec((N_REL, DB, D), lambda i: (0, i, 0)),
            pl.BlockSpec((DB, D), lambda i: (i, 0)),
            pl.BlockSpec((DB, D), lambda i: (i, 0)),
        ],
        out_shape=[
            jax.ShapeDtypeStruct((N_REL, D, D), jnp.bfloat16),
            jax.ShapeDtypeStruct((D, D), jnp.bfloat16),
            jax.ShapeDtypeStruct((D, D), jnp.bfloat16),
        ],
    )(comp, w_bases, w_root, gat_w)

    sec = jax.ShapeDtypeStruct((B, D), jnp.float32)
    secc = jax.ShapeDtypeStruct((D, B * C), jnp.float32)
    oft = jax.ShapeDtypeStruct((L, D, B * C), jnp.float32)
    o1t, o1e, o1cT, o2t, o2e, o2cT, out_final_t = pl.pallas_call(
        _gcn_kernel,
        grid=(B // GB,),
        in_specs=[
            pl.BlockSpec((GB, D), lambda i: (i, 0)),
            pl.BlockSpec((GB, D), lambda i: (i, 0)),
            pl.BlockSpec((GB * C, D), lambda i: (i, 0)),
            pl.BlockSpec((GB, N_REL * R, R), lambda i: (i, 0, 0)),
            pl.BlockSpec((N_REL, D, D), lambda i: (0, 0, 0)),
            pl.BlockSpec((D, D), lambda i: (0, 0)),
            pl.BlockSpec((D, D), lambda i: (0, 0)),
            pl.BlockSpec((4, D), lambda i: (0, 0)),
        ],
        out_specs=[
            pl.BlockSpec((GB, D), lambda i: (i, 0)),
            pl.BlockSpec((GB, D), lambda i: (i, 0)),
            pl.BlockSpec((D, GB * C), lambda i: (0, i)),
            pl.BlockSpec((GB, D), lambda i: (i, 0)),
            pl.BlockSpec((GB, D), lambda i: (i, 0)),
            pl.BlockSpec((D, GB * C), lambda i: (0, i)),
            pl.BlockSpec((L, D, GB * C), lambda i: (0, 0, i)),
        ],
        out_shape=[sec, sec, secc, sec, sec, secc, oft],
    )(target_node, emotion_node, cause_node, mb_arr, w_rel, wroot_b,
      gatw_b, vecs)

    out_1 = jnp.concatenate([o1t, o1e, o1cT.T], axis=0)
    out_2 = jnp.concatenate([o2t, o2e, o2cT.T], axis=0)
    out_final = jnp.transpose(out_final_t, (2, 0, 1))
    return (out_final, out_1, out_2)
